# baseline (speedup 1.0000x reference)
import functools

import jax
import jax.numpy as jnp
from jax import lax
from jax.experimental import pallas as pl
from jax.experimental.pallas import tpu as pltpu

N = 10000
E = 160000
D = 256


_BR = 1000


def _layernorm_in(h, w, b, eps=1e-5):
    mu = jnp.mean(h, axis=-1, keepdims=True)
    var = jnp.mean((h - mu) ** 2, axis=-1, keepdims=True)
    return (h - mu) / jnp.sqrt(var + eps) * w + b


def _dense_body(agg_ref, x_ref, wrel_ref, brel_ref, wroot_ref, lnw_ref,
                lnb_ref, wlin_ref, blin_ref, out_ref):
    agg = agg_ref[...]
    agg = jnp.where(jnp.isfinite(agg), agg, 0.0)
    x = x_ref[...]
    h = (lax.dot_general(agg, wrel_ref[...], (((1,), (1,)), ((), ())),
                         preferred_element_type=jnp.float32)
         + brel_ref[...][None, :]
         + lax.dot_general(x, wroot_ref[...], (((1,), (1,)), ((), ())),
                           preferred_element_type=jnp.float32))
    h = jax.nn.gelu(h)
    h = h + x
    h = _layernorm_in(h, lnw_ref[...][None, :], lnb_ref[...][None, :])
    h2 = lax.dot_general(h, wlin_ref[...], (((1,), (1,)), ((), ())),
                         preferred_element_type=jnp.float32) + blin_ref[...][None, :]
    h2 = jax.nn.gelu(h2) + h
    out_ref[...] = _layernorm_in(h2, lnw_ref[...][None, :], lnb_ref[...][None, :])


def _dense_chain(agg, x, W_rel, b_rel, W_root, ln_w, ln_b, W_lin, b_lin):
    full = lambda s: pl.BlockSpec(s, lambda i: (0,) * len(s))
    return pl.pallas_call(
        _dense_body,
        grid=(N // _BR,),
        in_specs=[
            pl.BlockSpec((_BR, D), lambda i: (i, 0)),
            pl.BlockSpec((_BR, D), lambda i: (i, 0)),
            full((D, D)), full((D,)), full((D, D)),
            full((D,)), full((D,)), full((D, D)), full((D,)),
        ],
        out_specs=pl.BlockSpec((_BR, D), lambda i: (i, 0)),
        out_shape=jax.ShapeDtypeStruct((N, D), jnp.float32),
    )(agg, x, W_rel, b_rel, W_root, ln_w, ln_b, W_lin, b_lin)


from jax.experimental.pallas import tpu_sc as plsc

_NC = 2
_NS = 16
_NW = _NC * _NS
_R = 320
_C = 3200
_G = 32
_L = 16


def _f16(v, dtype=jnp.int32):
    return jnp.full((_L,), v, dtype)


def _seg_max_body(x_hbm, src_hbm, dst_hbm, ew_hbm, out_hbm,
                  dst_v, src_v, w_v, lsrc, lw, ldst, gidx, rows_v, acc,
                  sem_c, sem_g):
    wid = lax.axis_index("s") * _NC + lax.axis_index("c")
    base = wid * _R
    base_v = jnp.full((_L,), base, jnp.int32)
    end_v = base_v + _R
    iota = lax.iota(jnp.int32, _L)
    ninf = jnp.full((_L,), -jnp.inf, jnp.float32)
    NCH = E // _C

    @plsc.parallel_loop(0, _R, 1, unroll=4)
    def init_row(i):
        rb = i * D
        for j in range(D // _L):
            plsc.store_scatter(acc, [rb + iota + j * _L], ninf)

    @plsc.parallel_loop(0, _C // _L, 1, unroll=4)
    def init_lsrc(i):
        plsc.store_scatter(lsrc, [iota + i * _L], _f16(0))

    def chunk_copies(c, pb):
        off = pl.multiple_of(c * _C, 8)
        return [
            pltpu.make_async_copy(dst_hbm.at[pl.ds(off, _C)], dst_v.at[pb], sem_c),
            pltpu.make_async_copy(src_hbm.at[pl.ds(off, _C)], src_v.at[pb], sem_c),
            pltpu.make_async_copy(ew_hbm.at[pl.ds(off, _C)], w_v.at[pb], sem_c),
        ]

    for d in chunk_copies(0, 0):
        d.start()

    def do_chunk(c, _):
        pb = c & 1
        for d in chunk_copies(c, pb):
            d.wait()

        @pl.when(c + 1 < NCH)
        def _():
            for d in chunk_copies(c + 1, (c + 1) & 1):
                d.start()

        def scan_group(i, cnt_vec):
            for u in range(2):
                s = i * (2 * _L) + u * _L
                vd = dst_v[pb, pl.ds(s, _L)]
                m = (vd >= base_v) & (vd < end_v)
                mi = m.astype(jnp.int32)
                pos = cnt_vec + plsc.cumsum(mi) - 1
                plsc.store_scatter(lsrc, [pos], src_v[pb, pl.ds(s, _L)], mask=m)
                plsc.store_scatter(lw, [pos], w_v[pb, pl.ds(s, _L)], mask=m)
                plsc.store_scatter(ldst, [pos], (vd - base_v) * D, mask=m)
                cnt_vec = cnt_vec + plsc.all_reduce_population_count(m)
            return cnt_vec
        cnt_vec = plsc.parallel_loop(
            0, _C // (2 * _L), 1, unroll=2, carry=_f16(0))(scan_group)
        cnt = lax.reduce_max(cnt_vec, (0,))

        nb = (cnt + _G - 1) >> 5

        def fill_gidx(b):
            gb = (b & 1) * _G
            boff = pl.multiple_of(b * _G, _G)
            for k in range(_G // _L):
                gidx[pl.ds(gb + k * _L, _L)] = lsrc[pl.ds(boff + k * _L, _L)]

        def gdesc(b):
            gb = (b & 1) * _G
            return pltpu.make_async_copy(
                x_hbm.at[gidx.at[pl.ds(gb, _G)]],
                rows_v.at[pl.ds(gb, _G)], sem_g)

        @pl.when(nb > 0)
        def _():
            fill_gidx(0)
            gdesc(0).start()

        def do_batch(b, _):
            gdesc(b).wait()

            @pl.when(b + 1 < nb)
            def _():
                fill_gidx(b + 1)
                gdesc(b + 1).start()

            rmax = jnp.minimum(cnt - b * _G, _G)
            rbase = (b & 1) * _G
            boff = pl.multiple_of(b * _G, _G)
            NJ = D // _L

            def one_edge(r):
                e = _f16(0) + (boff + r)
                wb = plsc.load_gather(lw, [e])
                db = plsc.load_gather(ldst, [e]) + iota
                cur = [plsc.load_gather(acc, [db + j * _L]) for j in range(NJ)]
                val = [rows_v[rbase + r, pl.ds(j * _L, _L)] * wb
                       for j in range(NJ)]
                for j in range(NJ):
                    plsc.store_scatter(acc, [db + j * _L],
                                       jnp.maximum(cur[j], val[j]))

            def do_pair(q, _):
                r0 = q * 2
                e0 = _f16(0) + (boff + r0)
                e1 = e0 + 1
                w0 = plsc.load_gather(lw, [e0])
                w1 = plsc.load_gather(lw, [e1])
                d0 = plsc.load_gather(ldst, [e0]) + iota
                d1 = plsc.load_gather(ldst, [e1]) + iota
                diff = lax.reduce_max(jnp.abs(d0 - d1), (0,))
                val0 = [rows_v[rbase + r0, pl.ds(j * _L, _L)] * w0
                        for j in range(NJ)]
                val1 = [rows_v[rbase + r0 + 1, pl.ds(j * _L, _L)] * w1
                        for j in range(NJ)]

                @pl.when(diff != 0)
                def _():
                    cur0 = [plsc.load_gather(acc, [d0 + j * _L])
                            for j in range(NJ)]
                    cur1 = [plsc.load_gather(acc, [d1 + j * _L])
                            for j in range(NJ)]
                    for j in range(NJ):
                        plsc.store_scatter(acc, [d0 + j * _L],
                                           jnp.maximum(cur0[j], val0[j]))
                    for j in range(NJ):
                        plsc.store_scatter(acc, [d1 + j * _L],
                                           jnp.maximum(cur1[j], val1[j]))

                @pl.when(diff == 0)
                def _():
                    cur0 = [plsc.load_gather(acc, [d0 + j * _L])
                            for j in range(NJ)]
                    for j in range(NJ):
                        plsc.store_scatter(
                            acc, [d0 + j * _L],
                            jnp.maximum(cur0[j],
                                        jnp.maximum(val0[j], val1[j])))
                return 0
            lax.fori_loop(0, rmax >> 1, do_pair, 0)

            @pl.when((rmax & 1) == 1)
            def _():
                one_edge(rmax - 1)
            return 0
        lax.fori_loop(0, nb, do_batch, 0)
        return 0

    lax.fori_loop(0, NCH, do_chunk, 0)

    pltpu.sync_copy(acc, out_hbm.at[pl.ds(base * D, _R * D)])


@functools.partial(jax.jit, static_argnums=())
def _segment_max(x, src, dst, ew):
    mesh = plsc.VectorSubcoreMesh(core_axis_name="c", subcore_axis_name="s")
    f = pl.kernel(
        _seg_max_body,
        out_type=jax.ShapeDtypeStruct((_NW * _R * D,), jnp.float32),
        mesh=mesh,
        compiler_params=pltpu.CompilerParams(use_tc_tiling_on_sc=False,
                                             needs_layout_passes=False),
        scratch_types=[
            pltpu.VMEM((2, _C), jnp.int32),
            pltpu.VMEM((2, _C), jnp.int32),
            pltpu.VMEM((2, _C), jnp.float32),
            pltpu.VMEM((_C,), jnp.int32),
            pltpu.VMEM((_C,), jnp.float32),
            pltpu.VMEM((_C,), jnp.int32),
            pltpu.VMEM((2 * _G,), jnp.int32),
            pltpu.VMEM((2 * _G, D), jnp.float32),
            pltpu.VMEM((_R * D,), jnp.float32),
            pltpu.SemaphoreType.DMA,
            pltpu.SemaphoreType.DMA,
        ],
    )
    return f(x, src, dst, ew).reshape(_NW * _R, D)


def kernel(x, edge_index, edge_weight, W_rel, b_rel, W_root, ln_w, ln_b,
           W_lin, b_lin):
    agg = _segment_max(x, edge_index[0], edge_index[1], edge_weight)[:N]
    h2 = _dense_chain(agg, x, W_rel, b_rel, W_root, ln_w, ln_b, W_lin, b_lin)
    return (h2, edge_weight)


# speedup vs baseline: 1.0277x; 1.0277x over previous
import functools

import jax
import jax.numpy as jnp
from jax import lax
from jax.experimental import pallas as pl
from jax.experimental.pallas import tpu as pltpu

N = 10000
E = 160000
D = 256


_BR = 1000


def _layernorm_in(h, w, b, eps=1e-5):
    mu = jnp.mean(h, axis=-1, keepdims=True)
    var = jnp.mean((h - mu) ** 2, axis=-1, keepdims=True)
    return (h - mu) / jnp.sqrt(var + eps) * w + b


def _dense_body(agg_ref, x_ref, wrel_ref, brel_ref, wroot_ref, lnw_ref,
                lnb_ref, wlin_ref, blin_ref, out_ref):
    agg = agg_ref[...]
    agg = jnp.where(jnp.isfinite(agg), agg, 0.0)
    x = x_ref[...]
    h = (lax.dot_general(agg, wrel_ref[...], (((1,), (1,)), ((), ())),
                         preferred_element_type=jnp.float32)
         + brel_ref[...][None, :]
         + lax.dot_general(x, wroot_ref[...], (((1,), (1,)), ((), ())),
                           preferred_element_type=jnp.float32))
    h = jax.nn.gelu(h)
    h = h + x
    h = _layernorm_in(h, lnw_ref[...][None, :], lnb_ref[...][None, :])
    h2 = lax.dot_general(h, wlin_ref[...], (((1,), (1,)), ((), ())),
                         preferred_element_type=jnp.float32) + blin_ref[...][None, :]
    h2 = jax.nn.gelu(h2) + h
    out_ref[...] = _layernorm_in(h2, lnw_ref[...][None, :], lnb_ref[...][None, :])


def _dense_chain(agg, x, W_rel, b_rel, W_root, ln_w, ln_b, W_lin, b_lin):
    full = lambda s: pl.BlockSpec(s, lambda i: (0,) * len(s))
    return pl.pallas_call(
        _dense_body,
        grid=(N // _BR,),
        in_specs=[
            pl.BlockSpec((_BR, D), lambda i: (i, 0)),
            pl.BlockSpec((_BR, D), lambda i: (i, 0)),
            full((D, D)), full((D,)), full((D, D)),
            full((D,)), full((D,)), full((D, D)), full((D,)),
        ],
        out_specs=pl.BlockSpec((_BR, D), lambda i: (i, 0)),
        out_shape=jax.ShapeDtypeStruct((N, D), jnp.float32),
    )(agg, x, W_rel, b_rel, W_root, ln_w, ln_b, W_lin, b_lin)


from jax.experimental.pallas import tpu_sc as plsc

_NC = 2
_NS = 16
_NW = _NC * _NS
_R = 320
_C = 3200
_G = 32
_L = 16


def _f16(v, dtype=jnp.int32):
    return jnp.full((_L,), v, dtype)


def _seg_max_body(x_hbm, src_hbm, dst_hbm, ew_hbm, out_hbm,
                  dst_v, src_v, w_v, lsrc, lw, ldst, gidx, rows_v, acc,
                  sem_c, sem_g):
    wid = lax.axis_index("s") * _NC + lax.axis_index("c")
    base = wid * _R
    base_v = jnp.full((_L,), base, jnp.int32)
    end_v = base_v + _R
    iota = lax.iota(jnp.int32, _L)
    ninf = jnp.full((_L,), -jnp.inf, jnp.float32)
    NCH = E // _C

    @plsc.parallel_loop(0, _R, 1, unroll=4)
    def init_row(i):
        rb = i * D
        for j in range(D // _L):
            plsc.store_scatter(acc, [rb + iota + j * _L], ninf)

    @plsc.parallel_loop(0, _C // _L, 1, unroll=4)
    def init_lsrc(i):
        plsc.store_scatter(lsrc, [iota + i * _L], _f16(0))

    def chunk_copies(c, pb):
        off = pl.multiple_of(c * _C, 8)
        return [
            pltpu.make_async_copy(dst_hbm.at[pl.ds(off, _C)], dst_v.at[pb], sem_c),
            pltpu.make_async_copy(src_hbm.at[pl.ds(off, _C)], src_v.at[pb], sem_c),
            pltpu.make_async_copy(ew_hbm.at[pl.ds(off, _C)], w_v.at[pb], sem_c),
        ]

    for d in chunk_copies(0, 0):
        d.start()

    def do_chunk(c, _):
        pb = c & 1
        for d in chunk_copies(c, pb):
            d.wait()

        @pl.when(c + 1 < NCH)
        def _():
            for d in chunk_copies(c + 1, (c + 1) & 1):
                d.start()

        def scan_group(i, cnt_vec):
            for u in range(2):
                s = i * (2 * _L) + u * _L
                vd = dst_v[pb, pl.ds(s, _L)]
                m = (vd >= base_v) & (vd < end_v)
                mi = m.astype(jnp.int32)
                pos = cnt_vec + plsc.cumsum(mi) - 1
                plsc.store_scatter(lsrc, [pos], src_v[pb, pl.ds(s, _L)], mask=m)
                plsc.store_scatter(lw, [pos], w_v[pb, pl.ds(s, _L)], mask=m)
                plsc.store_scatter(ldst, [pos], (vd - base_v) * D, mask=m)
                cnt_vec = cnt_vec + plsc.all_reduce_population_count(m)
            return cnt_vec
        cnt_vec = plsc.parallel_loop(
            0, _C // (2 * _L), 1, unroll=2, carry=_f16(0))(scan_group)
        cnt = lax.reduce_max(cnt_vec, (0,))

        nb = (cnt + _G - 1) >> 5

        def fill_gidx(b):
            gb = (b & 1) * _G
            boff = pl.multiple_of(b * _G, _G)
            for k in range(_G // _L):
                gidx[pl.ds(gb + k * _L, _L)] = lsrc[pl.ds(boff + k * _L, _L)]

        def gdesc(b):
            gb = (b & 1) * _G
            return pltpu.make_async_copy(
                x_hbm.at[gidx.at[pl.ds(gb, _G)]],
                rows_v.at[pl.ds(gb, _G)], sem_g)

        @pl.when(nb > 0)
        def _():
            fill_gidx(0)
            gdesc(0).start()

        def do_batch(b, _):
            gdesc(b).wait()

            @pl.when(b + 1 < nb)
            def _():
                fill_gidx(b + 1)
                gdesc(b + 1).start()

            rmax = jnp.minimum(cnt - b * _G, _G)
            rbase = (b & 1) * _G
            boff = pl.multiple_of(b * _G, _G)
            NJ = D // _L

            def one_edge(r):
                e = _f16(0) + (boff + r)
                wb = plsc.load_gather(lw, [e])
                db = plsc.load_gather(ldst, [e]) + iota
                cur = [plsc.load_gather(acc, [db + j * _L]) for j in range(NJ)]
                val = [rows_v[rbase + r, pl.ds(j * _L, _L)] * wb
                       for j in range(NJ)]
                for j in range(NJ):
                    plsc.store_scatter(acc, [db + j * _L],
                                       jnp.maximum(cur[j], val[j]))

            def do_edge(r, _):
                one_edge(r)
                return 0
            lax.fori_loop(0, rmax, do_edge, 0)
            return 0
        lax.fori_loop(0, nb, do_batch, 0)
        return 0

    lax.fori_loop(0, NCH, do_chunk, 0)

    pltpu.sync_copy(acc, out_hbm.at[pl.ds(base * D, _R * D)])


@functools.partial(jax.jit, static_argnums=())
def _segment_max(x, src, dst, ew):
    mesh = plsc.VectorSubcoreMesh(core_axis_name="c", subcore_axis_name="s")
    f = pl.kernel(
        _seg_max_body,
        out_type=jax.ShapeDtypeStruct((_NW * _R * D,), jnp.float32),
        mesh=mesh,
        compiler_params=pltpu.CompilerParams(use_tc_tiling_on_sc=False,
                                             needs_layout_passes=False),
        scratch_types=[
            pltpu.VMEM((2, _C), jnp.int32),
            pltpu.VMEM((2, _C), jnp.int32),
            pltpu.VMEM((2, _C), jnp.float32),
            pltpu.VMEM((_C,), jnp.int32),
            pltpu.VMEM((_C,), jnp.float32),
            pltpu.VMEM((_C,), jnp.int32),
            pltpu.VMEM((2 * _G,), jnp.int32),
            pltpu.VMEM((2 * _G, D), jnp.float32),
            pltpu.VMEM((_R * D,), jnp.float32),
            pltpu.SemaphoreType.DMA,
            pltpu.SemaphoreType.DMA,
        ],
    )
    return f(x, src, dst, ew).reshape(_NW * _R, D)


def kernel(x, edge_index, edge_weight, W_rel, b_rel, W_root, ln_w, ln_b,
           W_lin, b_lin):
    agg = _segment_max(x, edge_index[0], edge_index[1], edge_weight)[:N]
    h2 = _dense_chain(agg, x, W_rel, b_rel, W_root, ln_w, ln_b, W_lin, b_lin)
    return (h2, edge_weight)


# speedup vs baseline: 1.0402x; 1.0122x over previous
import functools

import jax
import jax.numpy as jnp
from jax import lax
from jax.experimental import pallas as pl
from jax.experimental.pallas import tpu as pltpu

N = 10000
E = 160000
D = 256


_BR = 1000


def _layernorm_in(h, w, b, eps=1e-5):
    mu = jnp.mean(h, axis=-1, keepdims=True)
    var = jnp.mean((h - mu) ** 2, axis=-1, keepdims=True)
    return (h - mu) / jnp.sqrt(var + eps) * w + b


def _dense_body(agg_ref, x_ref, wrel_ref, brel_ref, wroot_ref, lnw_ref,
                lnb_ref, wlin_ref, blin_ref, out_ref):
    agg = agg_ref[...].astype(jnp.float32)
    agg = jnp.where(jnp.isfinite(agg), agg, 0.0)
    x = x_ref[...]
    h = (lax.dot_general(agg, wrel_ref[...], (((1,), (1,)), ((), ())),
                         preferred_element_type=jnp.float32)
         + brel_ref[...][None, :]
         + lax.dot_general(x, wroot_ref[...], (((1,), (1,)), ((), ())),
                           preferred_element_type=jnp.float32))
    h = jax.nn.gelu(h)
    h = h + x
    h = _layernorm_in(h, lnw_ref[...][None, :], lnb_ref[...][None, :])
    h2 = lax.dot_general(h, wlin_ref[...], (((1,), (1,)), ((), ())),
                         preferred_element_type=jnp.float32) + blin_ref[...][None, :]
    h2 = jax.nn.gelu(h2) + h
    out_ref[...] = _layernorm_in(h2, lnw_ref[...][None, :], lnb_ref[...][None, :])


def _dense_chain(agg, x, W_rel, b_rel, W_root, ln_w, ln_b, W_lin, b_lin):
    full = lambda s: pl.BlockSpec(s, lambda i: (0,) * len(s))
    return pl.pallas_call(
        _dense_body,
        grid=(N // _BR,),
        in_specs=[
            pl.BlockSpec((_BR, D), lambda i: (i, 0)),
            pl.BlockSpec((_BR, D), lambda i: (i, 0)),
            full((D, D)), full((D,)), full((D, D)),
            full((D,)), full((D,)), full((D, D)), full((D,)),
        ],
        out_specs=pl.BlockSpec((_BR, D), lambda i: (i, 0)),
        out_shape=jax.ShapeDtypeStruct((N, D), jnp.float32),
    )(agg, x, W_rel, b_rel, W_root, ln_w, ln_b, W_lin, b_lin)


from jax.experimental.pallas import tpu_sc as plsc

_NC = 2
_NS = 16
_NW = _NC * _NS
_R = 320
_C = 4000
_G = 32
_L = 16


_DW = D // 2
_NINF2 = -8323200


def _f16(v, dtype=jnp.int32):
    return jnp.full((_L,), v, dtype)


def _seg_max_body(x_hbm, src_hbm, dst_hbm, ew_hbm, out_hbm,
                  dst_v, src_v, w_v, lsrc, lw, ldst, gidx, rows_v, acc,
                  sem_c, sem_g):
    wid = lax.axis_index("s") * _NC + lax.axis_index("c")
    base = wid * _R
    base_v = jnp.full((_L,), base, jnp.int32)
    end_v = base_v + _R
    iota = lax.iota(jnp.int32, _L)
    ninf = jnp.full((_L,), _NINF2, jnp.int32)
    NCH = E // _C

    @plsc.parallel_loop(0, _R, 1, unroll=4)
    def init_row(i):
        rb = i * _DW
        for j in range(_DW // _L):
            plsc.store_scatter(acc, [rb + iota + j * _L], ninf)

    @plsc.parallel_loop(0, _C // _L, 1, unroll=4)
    def init_lsrc(i):
        plsc.store_scatter(lsrc, [iota + i * _L], _f16(0))

    def chunk_copies(c, pb):
        off = pl.multiple_of(c * _C, 8)
        return [
            pltpu.make_async_copy(dst_hbm.at[pl.ds(off, _C)], dst_v.at[pb], sem_c),
            pltpu.make_async_copy(src_hbm.at[pl.ds(off, _C)], src_v.at[pb], sem_c),
            pltpu.make_async_copy(ew_hbm.at[pl.ds(off, _C)], w_v.at[pb], sem_c),
        ]

    for d in chunk_copies(0, 0):
        d.start()

    def do_chunk(c, _):
        pb = c & 1
        for d in chunk_copies(c, pb):
            d.wait()

        @pl.when(c + 1 < NCH)
        def _():
            for d in chunk_copies(c + 1, (c + 1) & 1):
                d.start()

        def scan_group(i, cnt_vec):
            for u in range(2):
                s = i * (2 * _L) + u * _L
                vd = dst_v[pb, pl.ds(s, _L)]
                m = (vd >= base_v) & (vd < end_v)
                mi = m.astype(jnp.int32)
                pos = cnt_vec + plsc.cumsum(mi) - 1
                plsc.store_scatter(lsrc, [pos], src_v[pb, pl.ds(s, _L)], mask=m)
                plsc.store_scatter(lw, [pos], w_v[pb, pl.ds(s, _L)], mask=m)
                plsc.store_scatter(ldst, [pos], (vd - base_v) * _DW, mask=m)
                cnt_vec = cnt_vec + plsc.all_reduce_population_count(m)
            return cnt_vec
        cnt_vec = plsc.parallel_loop(
            0, _C // (2 * _L), 1, unroll=2, carry=_f16(0))(scan_group)
        cnt = lax.reduce_max(cnt_vec, (0,))

        nb = (cnt + _G - 1) >> 5

        def fill_gidx(b):
            gb = (b & 1) * _G
            boff = pl.multiple_of(b * _G, _G)
            for k in range(_G // _L):
                gidx[pl.ds(gb + k * _L, _L)] = lsrc[pl.ds(boff + k * _L, _L)]

        def gdesc(b):
            gb = (b & 1) * _G
            return pltpu.make_async_copy(
                x_hbm.at[gidx.at[pl.ds(gb, _G)]],
                rows_v.at[pl.ds(gb, _G)], sem_g)

        @pl.when(nb > 0)
        def _():
            fill_gidx(0)
            gdesc(0).start()

        def do_batch(b, _):
            gdesc(b).wait()

            @pl.when(b + 1 < nb)
            def _():
                fill_gidx(b + 1)
                gdesc(b + 1).start()

            rmax = jnp.minimum(cnt - b * _G, _G)
            rbase = (b & 1) * _G
            boff = pl.multiple_of(b * _G, _G)
            NJ = _DW // _L

            def one_edge(r):
                e = _f16(0) + (boff + r)
                wb = plsc.load_gather(lw, [e])
                wb2 = plsc.pack(wb, wb, format=plsc.PackFormat.INTERLEAVED)
                db = plsc.load_gather(ldst, [e]) + iota
                cur = [plsc.load_gather(acc, [db + j * _L]) for j in range(NJ)]
                val = [plsc.bitcast(rows_v[rbase + r, pl.ds(j * _L, _L)],
                                    jnp.bfloat16) * wb2 for j in range(NJ)]
                for j in range(NJ):
                    mx = jnp.maximum(plsc.bitcast(cur[j], jnp.bfloat16),
                                     val[j])
                    plsc.store_scatter(acc, [db + j * _L],
                                       plsc.bitcast(mx, jnp.int32))

            def do_edge(r, _):
                one_edge(r)
                return 0
            lax.fori_loop(0, rmax, do_edge, 0)
            return 0
        lax.fori_loop(0, nb, do_batch, 0)
        return 0

    lax.fori_loop(0, NCH, do_chunk, 0)

    pltpu.sync_copy(acc, out_hbm.at[pl.ds(base * _DW, _R * _DW)])


@functools.partial(jax.jit, static_argnums=())
def _segment_max(x, src, dst, ew):
    mesh = plsc.VectorSubcoreMesh(core_axis_name="c", subcore_axis_name="s")
    f = pl.kernel(
        _seg_max_body,
        out_type=jax.ShapeDtypeStruct((_NW * _R * _DW,), jnp.int32),
        mesh=mesh,
        compiler_params=pltpu.CompilerParams(use_tc_tiling_on_sc=False,
                                             needs_layout_passes=False),
        scratch_types=[
            pltpu.VMEM((2, _C), jnp.int32),
            pltpu.VMEM((2, _C), jnp.int32),
            pltpu.VMEM((2, _C), jnp.float32),
            pltpu.VMEM((_C,), jnp.int32),
            pltpu.VMEM((_C,), jnp.float32),
            pltpu.VMEM((_C,), jnp.int32),
            pltpu.VMEM((2 * _G,), jnp.int32),
            pltpu.VMEM((2 * _G, _DW), jnp.int32),
            pltpu.VMEM((_R * _DW,), jnp.int32),
            pltpu.SemaphoreType.DMA,
            pltpu.SemaphoreType.DMA,
        ],
    )
    out_u = f(x, src, dst, ew)
    agg_bf = lax.bitcast_convert_type(
        out_u.reshape(_NW * _R, _DW), jnp.bfloat16)
    return agg_bf.reshape(_NW * _R, D)


def kernel(x, edge_index, edge_weight, W_rel, b_rel, W_root, ln_w, ln_b,
           W_lin, b_lin):
    x_u = lax.bitcast_convert_type(
        x.astype(jnp.bfloat16).reshape(N, _DW, 2), jnp.int32)
    agg = _segment_max(x_u, edge_index[0], edge_index[1], edge_weight)[:N]
    h2 = _dense_chain(agg, x, W_rel, b_rel, W_root, ln_w, ln_b, W_lin, b_lin)
    return (h2, edge_weight)
